# Initial kernel scaffold; baseline (speedup 1.0000x reference)
import jax
import jax.numpy as jnp
from jax.experimental import pallas as pl


def kernel(latent_ys, Wx, bx, Wu, bu, Ws, bs, level_weights, Wf, bf, gamma, beta):
    raise NotImplementedError("write your pallas kernel here")


# speedup vs baseline: 4.4375x; 4.4375x over previous
import functools

import jax
import jax.numpy as jnp
from jax.experimental import pallas as pl
from jax.experimental.pallas import tpu as pltpu

B = 64
T = 2048
D = 256
H = 64
NUM_LEVELS = 3


def _usum_body(x_ref, out_ref):
    j = pl.program_id(1)

    @pl.when(j == 0)
    def _():
        out_ref[...] = jnp.zeros_like(out_ref)

    out_ref[...] += jnp.sum(x_ref[...], axis=1)


def _u_sum(latent_ys, bb=8, tb=256):
    return pl.pallas_call(
        _usum_body,
        grid=(B // bb, T // tb),
        in_specs=[pl.BlockSpec((bb, tb, D), lambda i, j: (i, j, 0))],
        out_specs=pl.BlockSpec((bb, D), lambda i, j: (i, 0)),
        out_shape=jax.ShapeDtypeStruct((B, D), jnp.float32),
    )(latent_ys)


def _cross_body(xr_ref, us_ref, wx_ref, bx_ref, wu_ref, bu_ref, ws_ref,
                bs_ref, lw_ref, wf2_ref, bf_ref, out_ref):
    x_repr = xr_ref[...]
    u_repr = us_ref[...] * (1.0 / T)

    ex = jax.lax.dot_general(x_repr, wx_ref[...],
                             (((1,), (1,)), ((), ())),
                             preferred_element_type=jnp.float32) + bx_ref[...]
    eu = jax.lax.dot_general(u_repr, wu_ref[...],
                             (((1,), (1,)), ((), ())),
                             preferred_element_type=jnp.float32) + bu_ref[...]
    ws = ws_ref[...]
    sx = jax.lax.dot_general(ex, ws[:, :H],
                             (((1,), (1,)), ((), ())),
                             preferred_element_type=jnp.float32)
    su = jax.lax.dot_general(ws[:, H:], eu,
                             (((1,), (1,)), ((), ())),
                             preferred_element_type=jnp.float32)
    S = sx + su + bs_ref[0]

    row = jax.lax.broadcasted_iota(jnp.int32, (B, B), 0)
    col = jax.lax.broadcasted_iota(jnp.int32, (B, B), 1)
    ndiag = row != col
    S = jnp.where(ndiag, S, -1e30)
    S = S - jnp.max(S, axis=1, keepdims=True)
    E = jnp.exp(S)
    alpha = E / jnp.sum(E, axis=1, keepdims=True)

    lw0 = lw_ref[0]
    lw1 = lw_ref[1]
    lw2 = lw_ref[2]

    active = ndiag
    cross = jnp.zeros((B, D), jnp.float32)
    for l, lw in ((0, lw0), (1, lw1)):
        af = active.astype(jnp.float32)
        cnt = jnp.sum(af, axis=1, keepdims=True)
        ssum = jnp.sum(alpha * af, axis=1, keepdims=True)
        mean_val = jnp.where(cnt > 0, ssum / jnp.maximum(cnt, 1.0), 0.0)
        lower = active & (alpha <= mean_val)
        lf = lower.astype(jnp.float32)
        lcnt = jnp.sum(lf, axis=1, keepdims=True)
        lmean = jnp.dot(lf, u_repr,
                        preferred_element_type=jnp.float32) / jnp.maximum(lcnt, 1.0)
        cross += jnp.where(lcnt > 0, lw * lmean, 0.0)
        active = active & (~lower)

    af = active.astype(jnp.float32)
    acnt = jnp.sum(af, axis=1, keepdims=True)
    amean = jnp.dot(af, u_repr,
                    preferred_element_type=jnp.float32) / jnp.maximum(acnt, 1.0)
    alpha_m = jnp.where(ndiag, alpha, -1.0)
    rmax = jnp.max(alpha_m, axis=1, keepdims=True)
    at_max = alpha_m == rmax
    jidx = jnp.where(at_max, col, B)
    jstar = jnp.min(jidx, axis=1, keepdims=True)
    onehot = (col == jstar).astype(jnp.float32)
    fb = jnp.dot(onehot, u_repr, preferred_element_type=jnp.float32)
    cross += lw2 * jnp.where(acnt > 0, amean, fb)

    out_ref[...] = jax.lax.dot_general(
        cross, wf2_ref[...], (((1,), (1,)), ((), ())),
        preferred_element_type=jnp.float32) + bf_ref[...]


def _ccontrib(x_repr, u_sum, Wx, bx, Wu, bu, Ws, bs, level_weights, Wf2, bf):
    return pl.pallas_call(
        _cross_body,
        in_specs=[
            pl.BlockSpec(memory_space=pltpu.VMEM),
            pl.BlockSpec(memory_space=pltpu.VMEM),
            pl.BlockSpec(memory_space=pltpu.VMEM),
            pl.BlockSpec(memory_space=pltpu.VMEM),
            pl.BlockSpec(memory_space=pltpu.VMEM),
            pl.BlockSpec(memory_space=pltpu.VMEM),
            pl.BlockSpec(memory_space=pltpu.VMEM),
            pl.BlockSpec(memory_space=pltpu.SMEM),
            pl.BlockSpec(memory_space=pltpu.SMEM),
            pl.BlockSpec(memory_space=pltpu.VMEM),
            pl.BlockSpec(memory_space=pltpu.VMEM),
        ],
        out_shape=jax.ShapeDtypeStruct((B, D), jnp.float32),
    )(x_repr, u_sum, Wx, bx.reshape(1, H), Wu, bu.reshape(1, H), Ws,
      bs, level_weights, Wf2, bf.reshape(1, D))


def _fuse_body(x_ref, w_ref, c_ref, g_ref, b_ref, out_ref):
    x = x_ref[0]
    h = jax.lax.dot_general(x, w_ref[...], (((1,), (1,)), ((), ())),
                            preferred_element_type=jnp.float32)
    h = h + x + c_ref[0]
    mu = jnp.mean(h, axis=-1, keepdims=True)
    d = h - mu
    var = jnp.mean(d * d, axis=-1, keepdims=True)
    out_ref[0] = d / jnp.sqrt(var + 1e-5) * g_ref[...] + b_ref[...]


def _fuse(latent_ys, Wf1, ccontrib, gamma, beta, tb=512):
    return pl.pallas_call(
        _fuse_body,
        grid=(B, T // tb),
        in_specs=[
            pl.BlockSpec((1, tb, D), lambda i, j: (i, j, 0)),
            pl.BlockSpec((D, D), lambda i, j: (0, 0)),
            pl.BlockSpec((1, 1, D), lambda i, j: (i, 0, 0)),
            pl.BlockSpec((1, D), lambda i, j: (0, 0)),
            pl.BlockSpec((1, D), lambda i, j: (0, 0)),
        ],
        out_specs=pl.BlockSpec((1, tb, D), lambda i, j: (i, j, 0)),
        out_shape=jax.ShapeDtypeStruct((B, T, D), jnp.float32),
    )(latent_ys, Wf1, ccontrib.reshape(B, 1, D), gamma.reshape(1, D),
      beta.reshape(1, D))


@jax.jit
def kernel(latent_ys, Wx, bx, Wu, bu, Ws, bs, level_weights, Wf, bf, gamma,
           beta):
    x_repr = latent_ys[:, 0, :]
    u_sum = _u_sum(latent_ys)
    Wf1 = Wf[:, :D]
    Wf2 = Wf[:, D:]
    cc = _ccontrib(x_repr, u_sum, Wx, bx, Wu, bu, Ws, bs, level_weights,
                   Wf2, bf)
    return _fuse(latent_ys, Wf1, cc, gamma, beta)


# speedup vs baseline: 8.4052x; 1.8942x over previous
import jax
import jax.numpy as jnp
from jax.experimental import pallas as pl
from jax.experimental.pallas import tpu as pltpu

B = 64
T = 2048
D = 256
H = 64
NUM_LEVELS = 3


def _usum_body(x_ref, out_ref):
    j = pl.program_id(1)

    @pl.when(j == 0)
    def _():
        out_ref[...] = jnp.zeros_like(out_ref)

    out_ref[...] += jnp.sum(x_ref[...], axis=1)


def _u_sum(latent_ys, bb=8, tb=512):
    return pl.pallas_call(
        _usum_body,
        grid=(B // bb, T // tb),
        in_specs=[pl.BlockSpec((bb, tb, D), lambda i, j: (i, j, 0))],
        out_specs=pl.BlockSpec((bb, D), lambda i, j: (i, 0)),
        out_shape=jax.ShapeDtypeStruct((B, D), jnp.float32),
    )(latent_ys)


def _binning(x_repr, u_repr, wx, bx, wu, bu, ws, bs0, lw0, lw1, lw2, wf2, bf):
    ex = jax.lax.dot_general(x_repr, wx, (((1,), (1,)), ((), ())),
                             preferred_element_type=jnp.float32) + bx
    eu = jax.lax.dot_general(u_repr, wu, (((1,), (1,)), ((), ())),
                             preferred_element_type=jnp.float32) + bu
    sx = jax.lax.dot_general(ex, ws[:, :H], (((1,), (1,)), ((), ())),
                             preferred_element_type=jnp.float32)
    su = jax.lax.dot_general(ws[:, H:], eu, (((1,), (1,)), ((), ())),
                             preferred_element_type=jnp.float32)
    S = sx + su + bs0

    row = jax.lax.broadcasted_iota(jnp.int32, (B, B), 0)
    col = jax.lax.broadcasted_iota(jnp.int32, (B, B), 1)
    ndiag = row != col
    S = jnp.where(ndiag, S, -1e30)
    S = S - jnp.max(S, axis=1, keepdims=True)
    E = jnp.exp(S)
    alpha = E / jnp.sum(E, axis=1, keepdims=True)

    active = ndiag
    cross = jnp.zeros((B, D), jnp.float32)
    for lw in (lw0, lw1):
        af = active.astype(jnp.float32)
        cnt = jnp.sum(af, axis=1, keepdims=True)
        ssum = jnp.sum(alpha * af, axis=1, keepdims=True)
        mean_val = jnp.where(cnt > 0, ssum / jnp.maximum(cnt, 1.0), 0.0)
        lower = active & (alpha <= mean_val)
        lf = lower.astype(jnp.float32)
        lcnt = jnp.sum(lf, axis=1, keepdims=True)
        lmean = jnp.dot(lf, u_repr,
                        preferred_element_type=jnp.float32) / jnp.maximum(lcnt, 1.0)
        cross += jnp.where(lcnt > 0, lw * lmean, 0.0)
        active = active & (~lower)

    af = active.astype(jnp.float32)
    acnt = jnp.sum(af, axis=1, keepdims=True)
    amean = jnp.dot(af, u_repr,
                    preferred_element_type=jnp.float32) / jnp.maximum(acnt, 1.0)
    alpha_m = jnp.where(ndiag, alpha, -1.0)
    rmax = jnp.max(alpha_m, axis=1, keepdims=True)
    jidx = jnp.where(alpha_m == rmax, col, B)
    jstar = jnp.min(jidx, axis=1, keepdims=True)
    onehot = (col == jstar).astype(jnp.float32)
    fb = jnp.dot(onehot, u_repr, preferred_element_type=jnp.float32)
    cross += lw2 * jnp.where(acnt > 0, amean, fb)

    return jax.lax.dot_general(cross, wf2, (((1,), (1,)), ((), ())),
                               preferred_element_type=jnp.float32) + bf


def _fuse_body(xr_ref, us_ref, wx_ref, bx_ref, wu_ref, bu_ref, ws_ref,
               bs_ref, lw_ref, wf2_ref, bf_ref, x_ref, w_ref, g_ref, b_ref,
               out_ref, cc_ref):
    i = pl.program_id(0)

    @pl.when(i == 0)
    def _():
        cc_ref[...] = _binning(
            xr_ref[...], us_ref[...] * (1.0 / T), wx_ref[...], bx_ref[...],
            wu_ref[...], bu_ref[...], ws_ref[...], bs_ref[0], lw_ref[0],
            lw_ref[1], lw_ref[2], wf2_ref[...], bf_ref[...])

    x = x_ref[0]
    h = jax.lax.dot_general(x, w_ref[...], (((1,), (1,)), ((), ())),
                            preferred_element_type=jnp.float32)
    h = h + x + cc_ref[pl.ds(i, 1), :]
    mu = jnp.mean(h, axis=-1, keepdims=True)
    d = h - mu
    var = jnp.mean(d * d, axis=-1, keepdims=True)
    out_ref[0] = d * jax.lax.rsqrt(var + 1e-5) * g_ref[...] + b_ref[...]


def _fuse(x_repr, u_sum, Wx, bx, Wu, bu, Ws, bs, level_weights, Wf2, bf,
          latent_ys, Wf1, gamma, beta):
    cst = lambda shape: pl.BlockSpec(shape, lambda i: tuple(0 for _ in shape))
    return pl.pallas_call(
        _fuse_body,
        grid=(B,),
        in_specs=[
            cst((B, D)),
            cst((B, D)),
            cst((H, D)),
            cst((1, H)),
            cst((H, D)),
            cst((1, H)),
            cst((1, 2 * H)),
            pl.BlockSpec(memory_space=pltpu.SMEM),
            pl.BlockSpec(memory_space=pltpu.SMEM),
            cst((D, D)),
            cst((1, D)),
            pl.BlockSpec((1, T, D), lambda i: (i, 0, 0)),
            cst((D, D)),
            cst((1, D)),
            cst((1, D)),
        ],
        out_specs=pl.BlockSpec((1, T, D), lambda i: (i, 0, 0)),
        out_shape=jax.ShapeDtypeStruct((B, T, D), jnp.float32),
        scratch_shapes=[pltpu.VMEM((B, D), jnp.float32)],
    )(x_repr, u_sum, Wx, bx.reshape(1, H), Wu, bu.reshape(1, H), Ws, bs,
      level_weights, Wf2, bf.reshape(1, D), latent_ys, Wf1,
      gamma.reshape(1, D), beta.reshape(1, D))


@jax.jit
def kernel(latent_ys, Wx, bx, Wu, bu, Ws, bs, level_weights, Wf, bf, gamma,
           beta):
    x_repr = latent_ys[:, 0, :]
    u_sum = _u_sum(latent_ys)
    return _fuse(x_repr, u_sum, Wx, bx, Wu, bu, Ws, bs, level_weights,
                 Wf[:, D:], bf, latent_ys, Wf[:, :D], gamma, beta)
